# baseline (speedup 1.0000x reference)
import jax
import jax.numpy as jnp
from jax.experimental import pallas as pl
from jax.experimental.pallas import tpu as pltpu

NUM_CLASSES = 81
EPS = 1e-3

N = 5000
K = 7 * 7 * 256
H = 1024

BM = 1000
BK = 1792
NM = N // BM
NK = K // BK


def _head_kernel(x_ref, w1_ref, s1_ref, t1_ref, w2_ref, s2_ref, t2_ref,
                 wc_ref, bc_ref, wo_ref, bo_ref,
                 logit_ref, prob_ref, off_ref, acc_ref):
    k = pl.program_id(1)

    @pl.when(k == 0)
    def _zero():
        acc_ref[...] = jnp.zeros_like(acc_ref)

    acc_ref[...] += jnp.dot(x_ref[...], w1_ref[...],
                            preferred_element_type=jnp.float32)

    @pl.when(k == NK - 1)
    def _epilogue():
        y1 = jnp.maximum(acc_ref[...] * s1_ref[...] + t1_ref[...], 0.0)
        y2 = jnp.dot(y1, w2_ref[...], preferred_element_type=jnp.float32)
        y2 = jnp.maximum(y2 * s2_ref[...] + t2_ref[...], 0.0)
        logits = jnp.dot(y2, wc_ref[...],
                         preferred_element_type=jnp.float32) + bc_ref[...]
        logit_ref[...] = logits
        m = jnp.max(logits, axis=-1, keepdims=True)
        e = jnp.exp(logits - m)
        prob_ref[...] = e / jnp.sum(e, axis=-1, keepdims=True)
        off_ref[...] = jnp.dot(y2, wo_ref[...],
                               preferred_element_type=jnp.float32) + bo_ref[...]


def kernel(inputs, W1, b1, g1, be1, m1, v1, W2, b2, g2, be2, m2, v2, Wc, bc, Wo, bo):
    x = inputs.reshape(N, K)
    w1 = W1.reshape(K, H)
    w2 = W2.reshape(H, H)

    s1 = g1 * jax.lax.rsqrt(v1 + EPS)
    t1 = s1 * (b1 - m1) + be1
    s2 = g2 * jax.lax.rsqrt(v2 + EPS)
    t2 = s2 * (b2 - m2) + be2

    const = lambda bs: pl.BlockSpec(bs, lambda m, k: (0, 0))

    logit, prob, off = pl.pallas_call(
        _head_kernel,
        grid=(NM, NK),
        in_specs=[
            pl.BlockSpec((BM, BK), lambda m, k: (m, k)),
            pl.BlockSpec((BK, H), lambda m, k: (k, 0)),
            const((1, H)), const((1, H)),
            const((H, H)),
            const((1, H)), const((1, H)),
            const((H, NUM_CLASSES)), const((1, NUM_CLASSES)),
            const((H, 4 * NUM_CLASSES)), const((1, 4 * NUM_CLASSES)),
        ],
        out_specs=[
            pl.BlockSpec((BM, NUM_CLASSES), lambda m, k: (m, 0)),
            pl.BlockSpec((BM, NUM_CLASSES), lambda m, k: (m, 0)),
            pl.BlockSpec((BM, 4 * NUM_CLASSES), lambda m, k: (m, 0)),
        ],
        out_shape=[
            jax.ShapeDtypeStruct((N, NUM_CLASSES), jnp.float32),
            jax.ShapeDtypeStruct((N, NUM_CLASSES), jnp.float32),
            jax.ShapeDtypeStruct((N, 4 * NUM_CLASSES), jnp.float32),
        ],
        scratch_shapes=[pltpu.VMEM((BM, H), jnp.float32)],
        compiler_params=pltpu.CompilerParams(
            dimension_semantics=("parallel", "arbitrary"),
        ),
    )(x, w1,
      s1.reshape(1, H), t1.reshape(1, H),
      w2,
      s2.reshape(1, H), t2.reshape(1, H),
      Wc, bc.reshape(1, NUM_CLASSES),
      Wo, bo.reshape(1, 4 * NUM_CLASSES))

    return logit, prob, off.reshape(N, NUM_CLASSES, 4)


# speedup vs baseline: 1.0019x; 1.0019x over previous
import jax
import jax.numpy as jnp
from jax.experimental import pallas as pl
from jax.experimental.pallas import tpu as pltpu

NUM_CLASSES = 81
EPS = 1e-3

N = 5000
K = 7 * 7 * 256
H = 1024

BM = 1000
BK = 1792
NM = N // BM
NK = K // BK


def _head_kernel(x_ref, w1_ref, s1_ref, t1_ref, w2_ref, s2_ref, t2_ref,
                 wc_ref, bc_ref, wo_ref, bo_ref,
                 logit_ref, prob_ref, off_ref, acc_ref):
    k = pl.program_id(1)

    @pl.when(k == 0)
    def _zero():
        acc_ref[...] = jnp.zeros_like(acc_ref)

    acc_ref[...] += jnp.dot(x_ref[...].astype(jnp.bfloat16),
                            w1_ref[...].astype(jnp.bfloat16),
                            preferred_element_type=jnp.float32)

    @pl.when(k == NK - 1)
    def _epilogue():
        y1 = jnp.maximum(acc_ref[...] * s1_ref[...] + t1_ref[...], 0.0)
        y2 = jnp.dot(y1.astype(jnp.bfloat16),
                     w2_ref[...].astype(jnp.bfloat16),
                     preferred_element_type=jnp.float32)
        y2 = jnp.maximum(y2 * s2_ref[...] + t2_ref[...], 0.0)
        y2b = y2.astype(jnp.bfloat16)
        logits = jnp.dot(y2b, wc_ref[...].astype(jnp.bfloat16),
                         preferred_element_type=jnp.float32) + bc_ref[...]
        logit_ref[...] = logits
        m = jnp.max(logits, axis=-1, keepdims=True)
        e = jnp.exp(logits - m)
        prob_ref[...] = e / jnp.sum(e, axis=-1, keepdims=True)
        off_ref[...] = jnp.dot(y2b, wo_ref[...].astype(jnp.bfloat16),
                               preferred_element_type=jnp.float32) + bo_ref[...]


def kernel(inputs, W1, b1, g1, be1, m1, v1, W2, b2, g2, be2, m2, v2, Wc, bc, Wo, bo):
    x = inputs.reshape(N, K)
    w1 = W1.reshape(K, H)
    w2 = W2.reshape(H, H)

    s1 = g1 * jax.lax.rsqrt(v1 + EPS)
    t1 = s1 * (b1 - m1) + be1
    s2 = g2 * jax.lax.rsqrt(v2 + EPS)
    t2 = s2 * (b2 - m2) + be2

    const = lambda bs: pl.BlockSpec(bs, lambda m, k: (0, 0))

    logit, prob, off = pl.pallas_call(
        _head_kernel,
        grid=(NM, NK),
        in_specs=[
            pl.BlockSpec((BM, BK), lambda m, k: (m, k)),
            pl.BlockSpec((BK, H), lambda m, k: (k, 0)),
            const((1, H)), const((1, H)),
            const((H, H)),
            const((1, H)), const((1, H)),
            const((H, NUM_CLASSES)), const((1, NUM_CLASSES)),
            const((H, 4 * NUM_CLASSES)), const((1, 4 * NUM_CLASSES)),
        ],
        out_specs=[
            pl.BlockSpec((BM, NUM_CLASSES), lambda m, k: (m, 0)),
            pl.BlockSpec((BM, NUM_CLASSES), lambda m, k: (m, 0)),
            pl.BlockSpec((BM, 4 * NUM_CLASSES), lambda m, k: (m, 0)),
        ],
        out_shape=[
            jax.ShapeDtypeStruct((N, NUM_CLASSES), jnp.float32),
            jax.ShapeDtypeStruct((N, NUM_CLASSES), jnp.float32),
            jax.ShapeDtypeStruct((N, 4 * NUM_CLASSES), jnp.float32),
        ],
        scratch_shapes=[pltpu.VMEM((BM, H), jnp.float32)],
        compiler_params=pltpu.CompilerParams(
            dimension_semantics=("parallel", "arbitrary"),
        ),
    )(x, w1,
      s1.reshape(1, H), t1.reshape(1, H),
      w2,
      s2.reshape(1, H), t2.reshape(1, H),
      Wc, bc.reshape(1, NUM_CLASSES),
      Wo, bo.reshape(1, 4 * NUM_CLASSES))

    return logit, prob, off.reshape(N, NUM_CLASSES, 4)
